# baseline (speedup 1.0000x reference)
import jax
import jax.numpy as jnp
from jax import lax
from jax.experimental import pallas as pl
from jax.experimental.pallas import tpu as pltpu
from jax.experimental.pallas import tpu_sc as plsc

NC = 2
NS = 16
NW = NC * NS

B = 16384
F = 26
D = 16
V = 2600000
CB = 128
NJ = F * CB // 128
CHUNKS = B // (NW * CB)

TBLK = 1024
NBLK = -(-V // TBLK)


def _transpose_body(in_ref, out_ref):
    out_ref[...] = in_ref[...].T


def _relayout_table(embT):
    return pl.pallas_call(
        _transpose_body,
        grid=(NBLK,),
        in_specs=[pl.BlockSpec((D, TBLK), lambda i: (0, i))],
        out_specs=pl.BlockSpec((TBLK, D), lambda i: (i, 0)),
        out_shape=jax.ShapeDtypeStruct((V, D), jnp.float32),
    )(embT)


def _gather_body(idx_hbm, emb_hbm, out_hbm, idx_v, rows_v, sem):
    wid = lax.axis_index("s") * NC + lax.axis_index("c")

    for c in range(CHUNKS):
        pltpu.sync_copy(idx_hbm.at[wid, c], idx_v)

        def fire(j, carry):
            pltpu.async_copy(emb_hbm.at[idx_v.at[j]], rows_v.at[j], sem)
            return carry

        lax.fori_loop(0, NJ, fire, None)

        pltpu.make_async_copy(out_hbm.at[wid, c], rows_v, sem).wait()

        pltpu.sync_copy(rows_v, out_hbm.at[wid, c])


def _tokenizer(x, embeddings, bias, category_offsets):
    idx = (x + category_offsets[None, :]).reshape(NW, CHUNKS, NJ, 128)
    emb_lin = _relayout_table(embeddings.T)
    run = pl.kernel(
        _gather_body,
        out_type=jax.ShapeDtypeStruct((NW, CHUNKS, NJ, 128, D), jnp.float32),
        mesh=plsc.VectorSubcoreMesh(core_axis_name="c", subcore_axis_name="s"),
        scratch_types=[
            pltpu.VMEM((NJ, 128), jnp.int32),
            pltpu.VMEM((NJ, 128, D), jnp.float32),
            pltpu.SemaphoreType.DMA,
        ],
        compiler_params=pltpu.CompilerParams(use_tc_tiling_on_sc=False),
    )
    rows = run(idx, emb_lin)
    return rows.reshape(B, F, D) + bias[None, :, :]


def kernel(x, embeddings, bias, category_offsets):
    return jax.jit(_tokenizer)(x, embeddings, bias, category_offsets)


# speedup vs baseline: 1.6839x; 1.6839x over previous
import jax
import jax.numpy as jnp
from jax import lax
from jax.experimental import pallas as pl
from jax.experimental.pallas import tpu as pltpu
from jax.experimental.pallas import tpu_sc as plsc

NC = 2
NS = 16
NW = NC * NS

B = 16384
F = 26
D = 16
V = 2600000
CB = 128
NJ = F * CB // 128
CHUNKS = B // (NW * CB)

TBLK = 16384
NBLK = -(-V // TBLK)


def _transpose_body(in_ref, out_ref):
    out_ref[...] = in_ref[...].T


def _relayout_table(embT):
    return pl.pallas_call(
        _transpose_body,
        grid=(NBLK,),
        in_specs=[pl.BlockSpec((D, TBLK), lambda i: (0, i))],
        out_specs=pl.BlockSpec((TBLK, D), lambda i: (i, 0)),
        out_shape=jax.ShapeDtypeStruct((V, D), jnp.float32),
    )(embT)


def _gather_body(idx_hbm, emb_hbm, out_hbm, idx_v, rows_v, sem):
    wid = lax.axis_index("s") * NC + lax.axis_index("c")

    for c in range(CHUNKS):
        pltpu.sync_copy(idx_hbm.at[wid, c], idx_v)

        def fire(j, carry):
            pltpu.async_copy(emb_hbm.at[idx_v.at[j]], rows_v.at[j], sem)
            return carry

        lax.fori_loop(0, NJ, fire, None)

        pltpu.make_async_copy(out_hbm.at[wid, c], rows_v, sem).wait()

        pltpu.sync_copy(rows_v, out_hbm.at[wid, c])


def _tokenizer(x, embeddings, bias, category_offsets):
    idx = (x + category_offsets[None, :]).reshape(NW, CHUNKS, NJ, 128)
    emb_lin = _relayout_table(embeddings.T)
    run = pl.kernel(
        _gather_body,
        out_type=jax.ShapeDtypeStruct((NW, CHUNKS, NJ, 128, D), jnp.float32),
        mesh=plsc.VectorSubcoreMesh(core_axis_name="c", subcore_axis_name="s"),
        scratch_types=[
            pltpu.VMEM((NJ, 128), jnp.int32),
            pltpu.VMEM((NJ, 128, D), jnp.float32),
            pltpu.SemaphoreType.DMA,
        ],
        compiler_params=pltpu.CompilerParams(use_tc_tiling_on_sc=False),
    )
    rows = run(idx, emb_lin)
    return rows.reshape(B, F, D) + bias[None, :, :]


def kernel(x, embeddings, bias, category_offsets):
    return jax.jit(_tokenizer)(x, embeddings, bias, category_offsets)


# speedup vs baseline: 1.6868x; 1.0017x over previous
import jax
import jax.numpy as jnp
from jax import lax
from jax.experimental import pallas as pl
from jax.experimental.pallas import tpu as pltpu
from jax.experimental.pallas import tpu_sc as plsc

NC = 2
NS = 16
NW = NC * NS

B = 16384
F = 26
D = 16
V = 2600000
CB = 128
NJ = F * CB // 128
CHUNKS = B // (NW * CB)

TBLK = 16384
NBLK = -(-V // TBLK)


def _transpose_body(in_ref, out_ref):
    ident = jnp.eye(D, dtype=jnp.float32)
    out_ref[...] = lax.dot_general(
        in_ref[...], ident, (((0,), (0,)), ((), ())),
        preferred_element_type=jnp.float32,
    )


def _relayout_table(embT):
    return pl.pallas_call(
        _transpose_body,
        grid=(NBLK,),
        in_specs=[pl.BlockSpec((D, TBLK), lambda i: (0, i))],
        out_specs=pl.BlockSpec((TBLK, D), lambda i: (i, 0)),
        out_shape=jax.ShapeDtypeStruct((V, D), jnp.float32),
    )(embT)


def _gather_body(idx_hbm, emb_hbm, out_hbm, idx_v, rows_v, sem):
    wid = lax.axis_index("s") * NC + lax.axis_index("c")

    for c in range(CHUNKS):
        pltpu.sync_copy(idx_hbm.at[wid, c], idx_v)

        def fire(j, carry):
            pltpu.async_copy(emb_hbm.at[idx_v.at[j]], rows_v.at[j], sem)
            return carry

        lax.fori_loop(0, NJ, fire, None)

        pltpu.make_async_copy(out_hbm.at[wid, c], rows_v, sem).wait()

        pltpu.sync_copy(rows_v, out_hbm.at[wid, c])


def _tokenizer(x, embeddings, bias, category_offsets):
    idx = (x + category_offsets[None, :]).reshape(NW, CHUNKS, NJ, 128)
    emb_lin = _relayout_table(embeddings.T)
    run = pl.kernel(
        _gather_body,
        out_type=jax.ShapeDtypeStruct((NW, CHUNKS, NJ, 128, D), jnp.float32),
        mesh=plsc.VectorSubcoreMesh(core_axis_name="c", subcore_axis_name="s"),
        scratch_types=[
            pltpu.VMEM((NJ, 128), jnp.int32),
            pltpu.VMEM((NJ, 128, D), jnp.float32),
            pltpu.SemaphoreType.DMA,
        ],
        compiler_params=pltpu.CompilerParams(use_tc_tiling_on_sc=False),
    )
    rows = run(idx, emb_lin)
    return rows.reshape(B, F, D) + bias[None, :, :]


def kernel(x, embeddings, bias, category_offsets):
    return jax.jit(_tokenizer)(x, embeddings, bias, category_offsets)


# speedup vs baseline: 2.1679x; 1.2852x over previous
import jax
import jax.numpy as jnp
from jax import lax
from jax.experimental import pallas as pl
from jax.experimental.pallas import tpu as pltpu
from jax.experimental.pallas import tpu_sc as plsc

NC = 2
NS = 16
NW = NC * NS

B = 16384
F = 26
D = 16
V = 2600000
CB = 128
NJ = F * CB // 128
CHUNKS = B // (NW * CB)

TBLK = 16384
NBLK = -(-V // TBLK)


def _transpose_body(in_ref, out_ref):
    ident = jnp.eye(D, dtype=jnp.float32)
    out_ref[...] = lax.dot_general(
        in_ref[...], ident, (((0,), (0,)), ((), ())),
        preferred_element_type=jnp.float32,
    )


def _relayout_table(embT):
    return pl.pallas_call(
        _transpose_body,
        grid=(NBLK,),
        in_specs=[pl.BlockSpec((D, TBLK), lambda i: (0, i))],
        out_specs=pl.BlockSpec((TBLK, D), lambda i: (i, 0)),
        out_shape=jax.ShapeDtypeStruct((V, D), jnp.float32),
    )(embT)


EB = 2048
FD = F * D


def _epilogue_body(rows_ref, bias_ref, out_ref):
    y = rows_ref[...] + bias_ref[...]
    ident = jnp.eye(FD, dtype=jnp.float32)
    t = lax.dot_general(
        ident, y, (((0,), (1,)), ((), ())), preferred_element_type=jnp.float32
    )
    out_ref[...] = t.reshape(F, D, EB)


def _bias_and_relayout(rows, bias_flat):
    return pl.pallas_call(
        _epilogue_body,
        grid=(B // EB,),
        in_specs=[
            pl.BlockSpec((EB, FD), lambda j: (j, 0)),
            pl.BlockSpec((1, FD), lambda j: (0, 0)),
        ],
        out_specs=pl.BlockSpec((F, D, EB), lambda j: (0, 0, j)),
        out_shape=jax.ShapeDtypeStruct((F, D, B), jnp.float32),
    )(rows, bias_flat)


def _gather_body(idx_hbm, emb_hbm, out_hbm, idx_v, rows_v, sem):
    wid = lax.axis_index("s") * NC + lax.axis_index("c")

    for c in range(CHUNKS):
        pltpu.sync_copy(idx_hbm.at[wid, c], idx_v)

        def fire(j, carry):
            pltpu.async_copy(emb_hbm.at[idx_v.at[j]], rows_v.at[j], sem)
            return carry

        lax.fori_loop(0, NJ, fire, None)

        pltpu.make_async_copy(out_hbm.at[wid, c], rows_v, sem).wait()

        pltpu.sync_copy(rows_v, out_hbm.at[wid, c])


def _tokenizer(x, embeddings, bias, category_offsets):
    idx = (x + category_offsets[None, :]).reshape(NW, CHUNKS, NJ, 128)
    emb_lin = _relayout_table(embeddings.T)
    run = pl.kernel(
        _gather_body,
        out_type=jax.ShapeDtypeStruct((NW, CHUNKS, NJ, 128, D), jnp.float32),
        mesh=plsc.VectorSubcoreMesh(core_axis_name="c", subcore_axis_name="s"),
        scratch_types=[
            pltpu.VMEM((NJ, 128), jnp.int32),
            pltpu.VMEM((NJ, 128, D), jnp.float32),
            pltpu.SemaphoreType.DMA,
        ],
        compiler_params=pltpu.CompilerParams(use_tc_tiling_on_sc=False),
    )
    rows = run(idx, emb_lin)
    outT = _bias_and_relayout(rows.reshape(B, FD), bias.reshape(1, FD))
    return jnp.transpose(outT, (2, 0, 1))


def kernel(x, embeddings, bias, category_offsets):
    return jax.jit(_tokenizer)(x, embeddings, bias, category_offsets)


# speedup vs baseline: 10.2129x; 4.7110x over previous
import jax
import jax.numpy as jnp
from jax import lax
from jax.experimental import pallas as pl
from jax.experimental.pallas import tpu as pltpu
from jax.experimental.pallas import tpu_sc as plsc

NC = 2
NS = 16
NW = NC * NS

B = 16384
F = 26
D = 16
V = 2600000
CB = 128
NJ = F * CB // 128
CHUNKS = B // (NW * CB)

TBLK = 16384
NBLK = -(-V // TBLK)
SUB = TBLK // 8
V_PAD = NBLK * TBLK


def _transpose_body(in_ref, out_ref):
    x = in_ref[...]
    parts = [lax.slice(x, (0, k * SUB), (D, (k + 1) * SUB)) for k in range(8)]
    stacked = jnp.concatenate(parts, axis=0)
    ident = jnp.eye(128, dtype=jnp.float32)
    out_ref[...] = lax.dot_general(
        stacked, ident, (((0,), (0,)), ((), ())),
        preferred_element_type=jnp.float32,
    )


def _relayout_table(embT):
    return pl.pallas_call(
        _transpose_body,
        grid=(NBLK,),
        in_specs=[pl.BlockSpec((D, TBLK), lambda i: (0, i))],
        out_specs=pl.BlockSpec((SUB, 128), lambda i: (i, 0)),
        out_shape=jax.ShapeDtypeStruct((V_PAD // 8, 128), jnp.float32),
    )(embT)


def _packed_slot(r):
    return (r & ~(TBLK - 1)) + ((r & (SUB - 1)) << 3) + ((r >> 11) & 7)


EB = 2048
FD = F * D


def _epilogue_body(rows_ref, bias_ref, out_ref):
    y = rows_ref[...] + bias_ref[...]
    ident = jnp.eye(FD, dtype=jnp.float32)
    t = lax.dot_general(
        ident, y, (((0,), (1,)), ((), ())), preferred_element_type=jnp.float32
    )
    out_ref[...] = t.reshape(F, D, EB)


def _bias_and_relayout(rows, bias_flat):
    return pl.pallas_call(
        _epilogue_body,
        grid=(B // EB,),
        in_specs=[
            pl.BlockSpec((EB, FD), lambda j: (j, 0)),
            pl.BlockSpec((1, FD), lambda j: (0, 0)),
        ],
        out_specs=pl.BlockSpec((F, D, EB), lambda j: (0, 0, j)),
        out_shape=jax.ShapeDtypeStruct((F, D, B), jnp.float32),
    )(rows, bias_flat)


def _gather_body(idx_hbm, emb_hbm, out_hbm, idx_v, rows_v, sem):
    wid = lax.axis_index("s") * NC + lax.axis_index("c")

    for c in range(CHUNKS):
        pltpu.sync_copy(idx_hbm.at[wid, c], idx_v)

        def fire(j, carry):
            pltpu.async_copy(emb_hbm.at[idx_v.at[j]], rows_v.at[j], sem)
            return carry

        lax.fori_loop(0, NJ, fire, None)

        pltpu.make_async_copy(out_hbm.at[wid, c], rows_v, sem).wait()

        pltpu.sync_copy(rows_v, out_hbm.at[wid, c])


def _tokenizer(x, embeddings, bias, category_offsets):
    idx = _packed_slot(x + category_offsets[None, :]).reshape(NW, CHUNKS, NJ, 128)
    emb_lin = _relayout_table(embeddings.T).reshape(V_PAD, D)
    run = pl.kernel(
        _gather_body,
        out_type=jax.ShapeDtypeStruct((NW, CHUNKS, NJ, 128, D), jnp.float32),
        mesh=plsc.VectorSubcoreMesh(core_axis_name="c", subcore_axis_name="s"),
        scratch_types=[
            pltpu.VMEM((NJ, 128), jnp.int32),
            pltpu.VMEM((NJ, 128, D), jnp.float32),
            pltpu.SemaphoreType.DMA,
        ],
        compiler_params=pltpu.CompilerParams(use_tc_tiling_on_sc=False),
    )
    rows = run(idx, emb_lin)
    outT = _bias_and_relayout(rows.reshape(B, FD), bias.reshape(1, FD))
    return jnp.transpose(outT, (2, 0, 1))


def kernel(x, embeddings, bias, category_offsets):
    return jax.jit(_tokenizer)(x, embeddings, bias, category_offsets)
